# baseline (speedup 1.0000x reference)
import functools

import jax
import jax.numpy as jnp
from jax import lax
from jax.experimental import pallas as pl
from jax.experimental.pallas import tpu as pltpu
from jax.experimental.pallas import tpu_sc as plsc

HIDDEN = 128
NC = 2
NS = 16
NW = NC * NS

LANE = 128
K = 5
J = 3


def _make_gather(n_idx_rows):
    rows_per_w = n_idx_rows // NW
    n_groups = rows_per_w // K
    mesh = plsc.VectorSubcoreMesh(core_axis_name="c", subcore_axis_name="s")

    @functools.partial(
        pl.kernel,
        mesh=mesh,
        out_type=jax.ShapeDtypeStruct((n_idx_rows * LANE, HIDDEN), jnp.float32),
        scratch_types=(
            [pltpu.VMEM((LANE,), jnp.int32)] * K
            + [pltpu.VMEM((LANE, HIDDEN), jnp.float32)] * K
            + [pltpu.SemaphoreType.DMA] * (3 * K)
        ),
    )
    def gather_kernel(idx_hbm, table_hbm, out_hbm, *refs):
        wid = lax.axis_index("s") * NC + lax.axis_index("c")
        w_row0 = wid * rows_per_w
        IV = refs[0:K]
        RV = refs[K:2 * K]
        GS = refs[2 * K:3 * K]
        OS = refs[3 * K:4 * K]
        IS = refs[4 * K:5 * K]

        def drain_out(b):
            pltpu.make_async_copy(RV[b], out_hbm.at[pl.ds(0, LANE)],
                                  OS[b]).wait()

        def prefetch_idx(b, row0):
            pltpu.async_copy(idx_hbm.at[row0], IV[b], IS[b])

        def fire_gather(b):
            pltpu.make_async_copy(idx_hbm.at[0], IV[b], IS[b]).wait()
            pltpu.async_copy(table_hbm.at[IV[b]], RV[b], GS[b])

        for b in range(K):
            prefetch_idx(b, w_row0 + b)
        for b in range(J):
            fire_gather(b)

        def group(q, carry):
            for b in range(K):
                g = K * q + b
                bf = (b + J) % K
                if b < K - J:
                    pl.when(q >= 1)(lambda bf=bf: drain_out(bf))
                    fire_gather(bf)
                else:
                    drain_out(bf)
                    pl.when(q < n_groups - 1)(
                        lambda bf=bf: fire_gather(bf))
                pltpu.make_async_copy(table_hbm.at[IV[b]], RV[b],
                                      GS[b]).wait()
                pl.when(q < n_groups - 1)(
                    lambda b=b, g=g: prefetch_idx(b, w_row0 + g + K))
                pltpu.async_copy(
                    RV[b], out_hbm.at[pl.ds((w_row0 + g) * LANE, LANE)],
                    OS[b])
            return carry

        lax.fori_loop(0, n_groups, group, 0)
        for t in range(rows_per_w - (K - J), rows_per_w):
            drain_out(t % K)

    return gather_kernel


def kernel(batch_token_ids, token_embedding):
    b, s = batch_token_ids.shape
    n = b * s
    idx2d = batch_token_ids.reshape(n // LANE, LANE).astype(jnp.int32)
    out = _make_gather(n // LANE)(idx2d, token_embedding)
    return out.reshape(b, s, HIDDEN)
